# baseline (speedup 1.0000x reference)
import functools
import math

import jax
import jax.numpy as jnp
from jax import lax
from jax.experimental import pallas as pl
from jax.experimental.pallas import tpu as pltpu
from jax.experimental.pallas import tpu_sc as plsc

_EMB = 64
_SCALE = math.sqrt(_EMB)
_NC = 2
_NS = 16
_NW = _NC * _NS
_C = 128
_LANES = 16


@functools.partial(jax.jit, static_argnames=("n_chunks",))
def _embed(tok, table, n_chunks):
    bpw = n_chunks * _C
    b_total = _NW * bpw

    mesh = plsc.VectorSubcoreMesh(core_axis_name="c", subcore_axis_name="s")

    @functools.partial(
        pl.kernel,
        out_type=jax.ShapeDtypeStruct((b_total, _EMB), jnp.float32),
        mesh=mesh,
        compiler_params=pltpu.CompilerParams(use_tc_tiling_on_sc=False),
        scratch_types=[
            pltpu.VMEM((n_chunks, _C), jnp.int32),
            pltpu.VMEM((_C, _EMB), jnp.float32),
            pltpu.SemaphoreType.DMA,
        ],
    )
    def k(tok_hbm, table_hbm, out_hbm, idx_v, rows, gsem):
        wid = lax.axis_index("s") * _NC + lax.axis_index("c")
        base = wid * bpw
        pltpu.sync_copy(tok_hbm.at[wid], idx_v)

        def chunk(i, _):
            pltpu.async_copy(table_hbm.at[idx_v.at[i]], rows, gsem).wait()

            def srow(j, _):
                for l in range(_EMB // _LANES):
                    sl = (j, pl.ds(l * _LANES, _LANES))
                    rows[sl] = rows[sl] * _SCALE
                return 0

            lax.fori_loop(0, _C, srow, 0)
            pltpu.sync_copy(rows, out_hbm.at[pl.ds(base + i * _C, _C)])
            return 0

        lax.fori_loop(0, n_chunks, chunk, 0)

    return k(tok, table)


def kernel(tokens, table):
    b0, b1 = tokens.shape
    b_total = b0 * b1
    n_chunks = b_total // (_NW * _C)
    tok = tokens.astype(jnp.int32).reshape(_NW, n_chunks, _C)
    out = _embed(tok, table, n_chunks)
    return out.reshape(b0, b1, _EMB)


# speedup vs baseline: 1.1637x; 1.1637x over previous
import functools
import math

import jax
import jax.numpy as jnp
from jax import lax
from jax.experimental import pallas as pl
from jax.experimental.pallas import tpu as pltpu
from jax.experimental.pallas import tpu_sc as plsc

_EMB = 64
_SCALE = math.sqrt(_EMB)
_NC = 2
_NS = 16
_NW = _NC * _NS
_C = 128
_LANES = 16


@functools.partial(jax.jit, static_argnames=("n_chunks",))
def _embed(tok, table, n_chunks):
    bpw = n_chunks * _C
    b_total = _NW * bpw

    mesh = plsc.VectorSubcoreMesh(core_axis_name="c", subcore_axis_name="s")

    @functools.partial(
        pl.kernel,
        out_type=jax.ShapeDtypeStruct((b_total, _EMB), jnp.float32),
        mesh=mesh,
        compiler_params=pltpu.CompilerParams(use_tc_tiling_on_sc=False),
        scratch_types=[
            pltpu.VMEM((n_chunks, _C), jnp.int32),
            pltpu.VMEM((_C, _EMB), jnp.float32),
            pltpu.VMEM((_C, _EMB), jnp.float32),
            pltpu.SemaphoreType.DMA,
            pltpu.SemaphoreType.DMA,
            pltpu.SemaphoreType.DMA,
            pltpu.SemaphoreType.DMA,
        ],
    )
    def k(tok_hbm, table_hbm, out_hbm, idx_v, rows0, rows1, g0, g1, s0, s1):
        wid = lax.axis_index("s") * _NC + lax.axis_index("c")
        base = wid * bpw
        pltpu.sync_copy(tok_hbm.at[wid], idx_v)

        bufs = ((rows0, g0, s0), (rows1, g1, s1))

        pltpu.async_copy(table_hbm.at[idx_v.at[0]], rows0, g0)

        def pair(g, _):
            for b in range(2):
                i = g * 2 + b
                rows_b, gs_b, ss_b = bufs[b]
                rows_n, gs_n, ss_n = bufs[1 - b]

                @pl.when(i > 0)
                def _():
                    pltpu.make_async_copy(
                        rows_n, out_hbm.at[pl.ds(base + (i - 1) * _C, _C)], ss_n
                    ).wait()

                @pl.when(i + 1 < n_chunks)
                def _():
                    pltpu.async_copy(table_hbm.at[idx_v.at[i + 1]], rows_n, gs_n)

                pltpu.make_async_copy(
                    table_hbm.at[idx_v.at[i]], rows_b, gs_b
                ).wait()

                @plsc.parallel_loop(0, _C, step=1, unroll=8)
                def _(j):
                    for l in range(_EMB // _LANES):
                        sl = (j, pl.ds(l * _LANES, _LANES))
                        rows_b[sl] = rows_b[sl] * _SCALE

                pltpu.async_copy(
                    rows_b, out_hbm.at[pl.ds(base + i * _C, _C)], ss_b
                )
            return 0

        lax.fori_loop(0, n_chunks // 2, pair, 0)

        last = n_chunks - 1
        pltpu.make_async_copy(
            rows1, out_hbm.at[pl.ds(base + last * _C, _C)], s1
        ).wait()

    return k(tok, table)


def kernel(tokens, table):
    b0, b1 = tokens.shape
    b_total = b0 * b1
    n_chunks = b_total // (_NW * _C)
    tok = tokens.astype(jnp.int32).reshape(_NW, n_chunks, _C)
    out = _embed(tok, table, n_chunks)
    return out.reshape(b0, b1, _EMB)


# speedup vs baseline: 1.1864x; 1.0195x over previous
import functools
import math

import jax
import jax.numpy as jnp
from jax import lax
from jax.experimental import pallas as pl
from jax.experimental.pallas import tpu as pltpu
from jax.experimental.pallas import tpu_sc as plsc

_EMB = 64
_SCALE = math.sqrt(_EMB)
_NC = 2
_NS = 16
_NW = _NC * _NS
_L = 16
_CA = 128


@jax.jit
def _embed(tok, tably):
    b0, b1 = tok.shape
    rpw = b0 // _NW
    cb = b1 - _CA
    offs = [m * _L for m in range(b1 // _L)] + [b1 - _L]
    mesh = plsc.VectorSubcoreMesh(core_axis_name="c", subcore_axis_name="s")

    @functools.partial(
        pl.kernel,
        out_type=jax.ShapeDtypeStruct((b0, b1, _EMB), jnp.float32),
        mesh=mesh,
        scratch_types=[
            pltpu.VMEM((2, 8, b1), jnp.int32),
            pltpu.VMEM((2, 8, b1), jnp.int32),
            pltpu.VMEM((2, 8, b1), jnp.int32),
            pltpu.VMEM((_CA, 2 * _EMB), jnp.float32),
            pltpu.VMEM((_CA, 2 * _EMB), jnp.float32),
            pltpu.VMEM((cb, 2 * _EMB), jnp.float32),
            pltpu.VMEM((cb, 2 * _EMB), jnp.float32),
            pltpu.VMEM((b1, _EMB), jnp.float32),
            pltpu.VMEM((b1, _EMB), jnp.float32),
            pltpu.SemaphoreType.DMA,
            pltpu.SemaphoreType.DMA,
            pltpu.SemaphoreType.DMA,
            pltpu.SemaphoreType.DMA,
            pltpu.SemaphoreType.DMA,
            pltpu.SemaphoreType.DMA,
            pltpu.SemaphoreType.DMA,
        ],
    )
    def k(tok_hbm, tably_hbm, out_hbm, tokv, idxh, par64,
          ra0, ra1, rb0, rb1, ov0, ov1, tsem, ga0, ga1, gb0, gb1, s0, s1):
        wid = lax.axis_index("s") * _NC + lax.axis_index("c")
        base0 = wid * rpw

        def build(slot):
            for q in range(8):
                for o in offs:
                    t = tokv[slot, q, pl.ds(o, _L)]
                    hi = lax.shift_left(lax.shift_right_logical(t, 12), 11)
                    idxh[slot, q, pl.ds(o, _L)] = hi | (t & 2047)
                    par64[slot, q, pl.ds(o, _L)] = (
                        lax.shift_right_logical(t, 5) & _EMB)

        def stage_start(batch, slot):
            pltpu.async_copy(
                tok_hbm.at[pl.ds(base0 + batch * 8, 8)], tokv.at[slot], tsem
            )

        def stage_wait(batch, slot):
            pltpu.make_async_copy(
                tok_hbm.at[pl.ds(base0 + batch * 8, 8)], tokv.at[slot], tsem
            ).wait()

        def gstart(r, ra, rb, ga, gb):
            kb = (r // 8) % 2
            q = r % 8
            pltpu.async_copy(
                tably_hbm.at[idxh.at[kb, q, pl.ds(0, _CA)]], ra, ga)
            pltpu.async_copy(
                tably_hbm.at[idxh.at[kb, q, pl.ds(_CA, cb)]], rb, gb)

        def gwait(r, ra, rb, ga, gb):
            kb = (r // 8) % 2
            q = r % 8
            pltpu.make_async_copy(
                tably_hbm.at[idxh.at[kb, q, pl.ds(0, _CA)]], ra, ga).wait()
            pltpu.make_async_copy(
                tably_hbm.at[idxh.at[kb, q, pl.ds(_CA, cb)]], rb, gb).wait()

        pltpu.sync_copy(tok_hbm.at[pl.ds(base0, 8)], tokv.at[0])
        build(0)
        stage_start(1, 1)
        gstart(0, ra0, rb0, ga0, gb0)

        bufs = (
            (ra0, rb0, ov0, ga0, gb0, s0),
            (ra1, rb1, ov1, ga1, gb1, s1),
        )

        def pair(p, _):
            for b in range(2):
                r = p * 2 + b
                ra_b, rb_b, ov_b, ga_b, gb_b, ss_b = bufs[b]
                ra_n, rb_n, ov_n, ga_n, gb_n, ss_n = bufs[1 - b]
                kb = (r // 8) % 2
                q = r % 8

                if b == 0:
                    @pl.when((q == 0) & (r < rpw - 8))
                    def _():
                        m1 = r // 8 + 1
                        stage_wait(m1, 1 - kb)
                        build(1 - kb)

                        @pl.when(r < rpw - 16)
                        def _():
                            stage_start(r // 8 + 2, kb)

                @pl.when(r > 0)
                def _():
                    pltpu.make_async_copy(
                        ov_n, out_hbm.at[base0 + r - 1], ss_n
                    ).wait()

                @pl.when(r + 1 < rpw)
                def _():
                    gstart(r + 1, ra_n, rb_n, ga_n, gb_n)

                gwait(r, ra_b, rb_b, ga_b, gb_b)

                def sel_a(m, _):
                    pg = par64[kb, q, pl.ds(m * _L, _L)]
                    for i in range(_L):
                        j = m * _L + i
                        c = pg[i]
                        for l in range(_EMB // _L):
                            ov_b[j, pl.ds(l * _L, _L)] = (
                                ra_b[j, pl.ds(c + l * _L, _L)])
                    return 0

                def sel_b(m, _):
                    pg = par64[kb, q, pl.ds(_CA + m * _L, _L)]
                    for i in range(_L):
                        j = m * _L + i
                        c = pg[i]
                        for l in range(_EMB // _L):
                            ov_b[_CA + j, pl.ds(l * _L, _L)] = (
                                rb_b[j, pl.ds(c + l * _L, _L)])
                    return 0

                lax.fori_loop(0, _CA // _L, sel_a, 0)
                lax.fori_loop(0, cb // _L, sel_b, 0)
                mo = cb - _L
                pg = par64[kb, q, pl.ds(_CA + mo, _L)]
                for i in range(_L):
                    j = mo + i
                    c = pg[i]
                    for l in range(_EMB // _L):
                        ov_b[_CA + j, pl.ds(l * _L, _L)] = (
                            rb_b[j, pl.ds(c + l * _L, _L)])

                pltpu.async_copy(ov_b, out_hbm.at[base0 + r], ss_b)
            return 0

        lax.fori_loop(0, rpw // 2, pair, 0)

        pltpu.make_async_copy(
            ov1, out_hbm.at[base0 + rpw - 1], s1
        ).wait()

    return k(tok, tably)


_PRE_W = 4096


def _pre_body(x_ref, o_ref):
    h = _PRE_W // 2
    o_ref[:, 0:_EMB] = jnp.transpose(x_ref[:, 0:h]) * _SCALE
    o_ref[:, _EMB:2 * _EMB] = jnp.transpose(x_ref[:, h:_PRE_W]) * _SCALE


@jax.jit
def _pre_transpose(tt):
    emb, v = tt.shape
    grid = (v + _PRE_W - 1) // _PRE_W
    return pl.pallas_call(
        _pre_body,
        grid=(grid,),
        in_specs=[pl.BlockSpec((emb, _PRE_W), lambda i: (0, i))],
        out_specs=pl.BlockSpec((_PRE_W // 2, 2 * _EMB), lambda i: (i, 0)),
        out_shape=jax.ShapeDtypeStruct(
            (grid * _PRE_W // 2, 2 * emb), jnp.float32),
    )(tt)


def kernel(tokens, table):
    tably = _pre_transpose(table.T)
    return _embed(tokens.astype(jnp.int32), tably)
